# Initial kernel scaffold; baseline (speedup 1.0000x reference)
import jax
import jax.numpy as jnp
from jax.experimental import pallas as pl


def kernel(x, edge_index, y, batch, W1, b1, W2, b2, W3, b3, W4, b4, W5, b5, p1, p2, fcW1, fcb1, fcW2, fcb2):
    raise NotImplementedError("write your pallas kernel here")


# speedup vs baseline: 12.2909x; 12.2909x over previous
import functools

import jax
import jax.numpy as jnp
from jax import lax
from jax.experimental import pallas as pl
from jax.experimental.pallas import tpu as pltpu
from jax.experimental.pallas import tpu_sc as plsc

N = 10000
E = 320000
G = 64
F = 64
NP = 10240
TRASH = N

RW = 80
ER = E // RW
ERP = 4096
NW = 32
RPW = ERP // NW
RCHUNK = 32
NCHUNK = RPW // RCHUNK
STRIPE = NP // 16

RB = 1000
GRID = N // RB

_f32 = jnp.float32
_i32 = jnp.int32

_mesh = plsc.VectorSubcoreMesh(core_axis_name="c", subcore_axis_name="s")
_sc_params = pltpu.CompilerParams(use_tc_tiling_on_sc=False,
                                  needs_layout_passes=False)


def _wid(cid, sid):
    return cid * 16 + sid


@functools.partial(
    pl.kernel,
    out_type=jax.ShapeDtypeStruct((2, NP, 16), _f32),
    mesh=_mesh,
    compiler_params=_sc_params,
    scratch_types=[
        pltpu.VMEM((RCHUNK, RW), _i32),
        pltpu.VMEM((RW, 16), _f32),
        pltpu.VMEM((STRIPE, 16), _f32),
        pltpu.VMEM_SHARED((NP, 16), _f32),
    ],
)
def _sc_deg0(dst_h, degp_h, idxv, onesv, stripev, degacc):
    cid = lax.axis_index("c")
    sid = lax.axis_index("s")
    wid = _wid(cid, sid)

    e0 = (lax.broadcasted_iota(_i32, (16,), 0) == 0).astype(_f32)

    def fill16(i, _):
        onesv[i, :] = e0
        return 0

    lax.fori_loop(0, RW, fill16, 0)

    def zfill(i, _):
        stripev[i, :] = jnp.zeros((16,), _f32)
        return 0

    lax.fori_loop(0, STRIPE, zfill, 0)

    pltpu.sync_copy(stripev, degacc.at[pl.ds(sid * STRIPE, STRIPE)])
    plsc.subcore_barrier()

    def chunk(j, _):
        roff = wid * RPW + j * RCHUNK
        pltpu.sync_copy(dst_h.at[pl.ds(roff, RCHUNK)], idxv)

        def row(r, _):
            pltpu.sync_copy(onesv, degacc.at[idxv.at[r]], add=True)
            return 0

        lax.fori_loop(0, RCHUNK, row, 0)
        return 0

    lax.fori_loop(0, NCHUNK, chunk, 0)
    plsc.subcore_barrier()
    pltpu.sync_copy(degacc.at[pl.ds(sid * STRIPE, STRIPE)], stripev)
    pltpu.sync_copy(stripev, degp_h.at[cid, pl.ds(sid * STRIPE, STRIPE)])


@functools.partial(
    pl.kernel,
    out_type=[
        jax.ShapeDtypeStruct((ERP, RW), _i32),
        jax.ShapeDtypeStruct((2, NP, 16), _f32),
    ],
    mesh=_mesh,
    compiler_params=_sc_params,
    scratch_types=[
        pltpu.VMEM((NP,), _i32),
        pltpu.VMEM((RCHUNK, RW), _i32),
        pltpu.VMEM((RCHUNK, RW), _i32),
        pltpu.VMEM((RCHUNK, RW), _i32),
        pltpu.VMEM((RW, 16), _f32),
        pltpu.VMEM((STRIPE, 16), _f32),
        pltpu.VMEM_SHARED((NP, 16), _f32),
    ],
)
def _sc_prep(src_h, dstin_h, sel_h, dste_h, degp_h, selv, srcv, dstv, dstev,
             onesv, stripev, degacc):
    cid = lax.axis_index("c")
    sid = lax.axis_index("s")
    wid = _wid(cid, sid)

    e0 = (lax.broadcasted_iota(_i32, (16,), 0) == 0).astype(_f32)

    def fill16(i, _):
        onesv[i, :] = e0
        return 0

    lax.fori_loop(0, RW, fill16, 0)

    def zfill(i, _):
        stripev[i, :] = jnp.zeros((16,), _f32)
        return 0

    lax.fori_loop(0, STRIPE, zfill, 0)
    pltpu.sync_copy(sel_h, selv)

    pltpu.sync_copy(stripev, degacc.at[pl.ds(sid * STRIPE, STRIPE)])
    plsc.subcore_barrier()

    def chunk(j, _):
        roff = wid * RPW + j * RCHUNK
        pltpu.sync_copy(src_h.at[pl.ds(roff, RCHUNK)], srcv)
        pltpu.sync_copy(dstin_h.at[pl.ds(roff, RCHUNK)], dstv)

        def row(r, _):
            def vec(i, _):
                s16 = srcv[r, pl.ds(i * 16, 16)]
                d16 = dstv[r, pl.ds(i * 16, 16)]
                ssel = plsc.load_gather(selv, [s16])
                dsel = plsc.load_gather(selv, [d16])
                valid = (ssel > 0) & (dsel > 0)
                dstev[r, pl.ds(i * 16, 16)] = jnp.where(
                    valid, d16, jnp.full((16,), TRASH, _i32))
                return 0

            lax.fori_loop(0, RW // 16, vec, 0)
            pltpu.sync_copy(onesv, degacc.at[dstev.at[r]], add=True)
            return 0

        lax.fori_loop(0, RCHUNK, row, 0)
        pltpu.sync_copy(dstev, dste_h.at[pl.ds(roff, RCHUNK)])
        return 0

    lax.fori_loop(0, NCHUNK, chunk, 0)
    plsc.subcore_barrier()
    pltpu.sync_copy(degacc.at[pl.ds(sid * STRIPE, STRIPE)], stripev)
    pltpu.sync_copy(stripev, degp_h.at[cid, pl.ds(sid * STRIPE, STRIPE)])


@functools.partial(
    pl.kernel,
    out_type=jax.ShapeDtypeStruct((2, NP, F), _f32),
    mesh=_mesh,
    compiler_params=_sc_params,
    scratch_types=[
        pltpu.VMEM((RCHUNK, RW), _i32),
        pltpu.VMEM((RCHUNK, RW), _i32),
        pltpu.VMEM((RW, F), _f32),
        pltpu.SemaphoreType.DMA,
        pltpu.VMEM_SHARED((NP, F), _f32),
    ],
)
def _sc_agg(tab_h, src_h, dste_h, p_h, srcv, dstev, rows, sem, acc):
    cid = lax.axis_index("c")
    sid = lax.axis_index("s")
    wid = _wid(cid, sid)

    def zfill(i, _):
        r = i // (F // 16)
        cidx = (i % (F // 16)) * 16
        rows[r, pl.ds(cidx, 16)] = jnp.zeros((16,), _f32)
        return 0

    lax.fori_loop(0, RW * (F // 16), zfill, 0)

    def zstripe(t, _):
        pltpu.sync_copy(rows, acc.at[pl.ds(sid * STRIPE + t * RW, RW)])
        return 0

    lax.fori_loop(0, STRIPE // RW, zstripe, 0)
    plsc.subcore_barrier()

    def chunk(j, _):
        roff = wid * RPW + j * RCHUNK
        pltpu.sync_copy(src_h.at[pl.ds(roff, RCHUNK)], srcv)
        pltpu.sync_copy(dste_h.at[pl.ds(roff, RCHUNK)], dstev)

        def row(r, _):
            pltpu.async_copy(tab_h.at[srcv.at[r]], rows, sem).wait()
            pltpu.sync_copy(rows, acc.at[dstev.at[r]], add=True)
            return 0

        lax.fori_loop(0, RCHUNK, row, 0)
        return 0

    lax.fori_loop(0, NCHUNK, chunk, 0)
    plsc.subcore_barrier()

    def out(t, _):
        off = sid * STRIPE + t * RW
        pltpu.sync_copy(acc.at[pl.ds(off, RW)], rows)
        pltpu.sync_copy(rows, p_h.at[cid, pl.ds(off, RW)])
        return 0

    lax.fori_loop(0, STRIPE // RW, out, 0)


def _mm_scale_body(x_ref, w_ref, d0_ref, d1_ref, sc_ref, sl_ref, th_ref,
                   dinv_ref):
    deg = 1.0 + d0_ref[...] + d1_ref[...]
    dinv = lax.rsqrt(deg)
    xs = x_ref[...] * (sc_ref[...] * sl_ref[...])
    t = jnp.dot(xs, w_ref[...], preferred_element_type=_f32)
    th_ref[...] = t * dinv
    dinv_ref[...] = dinv


def _mm_scale(x, w, d0, d1, scol, slcol):
    k = x.shape[1]
    return pl.pallas_call(
        _mm_scale_body,
        grid=(GRID,),
        in_specs=[
            pl.BlockSpec((RB, k), lambda i: (i, 0)),
            pl.BlockSpec((k, F), lambda i: (0, 0)),
            pl.BlockSpec((RB, 1), lambda i: (i, 0)),
            pl.BlockSpec((RB, 1), lambda i: (i, 0)),
            pl.BlockSpec((RB, 1), lambda i: (i, 0)),
            pl.BlockSpec((RB, 1), lambda i: (i, 0)),
        ],
        out_specs=[
            pl.BlockSpec((RB, F), lambda i: (i, 0)),
            pl.BlockSpec((RB, 1), lambda i: (i, 0)),
        ],
        out_shape=[
            jax.ShapeDtypeStruct((N, F), _f32),
            jax.ShapeDtypeStruct((N, 1), _f32),
        ],
    )(x, w, d0, d1, scol, slcol)


def _combine_mm_body(p0_ref, p1_ref, th_ref, dinv_ref, b_ref, w_ref, h_ref,
                     th2_ref):
    dinv = dinv_ref[...]
    h = jnp.maximum(
        dinv * (p0_ref[...] + p1_ref[...] + th_ref[...]) + b_ref[...], 0.0)
    h_ref[...] = h
    th2_ref[...] = jnp.dot(h, w_ref[...], preferred_element_type=_f32) * dinv


def _combine_mm(p0, p1, th, dinv, b, w):
    return pl.pallas_call(
        _combine_mm_body,
        grid=(GRID,),
        in_specs=[
            pl.BlockSpec((RB, F), lambda i: (i, 0)),
            pl.BlockSpec((RB, F), lambda i: (i, 0)),
            pl.BlockSpec((RB, F), lambda i: (i, 0)),
            pl.BlockSpec((RB, 1), lambda i: (i, 0)),
            pl.BlockSpec((1, F), lambda i: (0, 0)),
            pl.BlockSpec((F, F), lambda i: (0, 0)),
        ],
        out_specs=[
            pl.BlockSpec((RB, F), lambda i: (i, 0)),
            pl.BlockSpec((RB, F), lambda i: (i, 0)),
        ],
        out_shape=[
            jax.ShapeDtypeStruct((N, F), _f32),
            jax.ShapeDtypeStruct((N, F), _f32),
        ],
    )(p0, p1, th, dinv, b, w)


def _combine_body(p0_ref, p1_ref, th_ref, dinv_ref, b_ref, h_ref):
    h_ref[...] = jnp.maximum(
        dinv_ref[...] * (p0_ref[...] + p1_ref[...] + th_ref[...])
        + b_ref[...], 0.0)


def _combine(p0, p1, th, dinv, b):
    return pl.pallas_call(
        _combine_body,
        grid=(GRID,),
        in_specs=[
            pl.BlockSpec((RB, F), lambda i: (i, 0)),
            pl.BlockSpec((RB, F), lambda i: (i, 0)),
            pl.BlockSpec((RB, F), lambda i: (i, 0)),
            pl.BlockSpec((RB, 1), lambda i: (i, 0)),
            pl.BlockSpec((1, F), lambda i: (0, 0)),
        ],
        out_specs=pl.BlockSpec((RB, F), lambda i: (i, 0)),
        out_shape=jax.ShapeDtypeStruct((N, F), _f32),
    )(p0, p1, th, dinv, b)


def _score_body(h_ref, p_ref, s_ref):
    p = p_ref[...]
    nrm = jnp.sqrt(jnp.sum(p * p)) + 1e-16
    s_ref[...] = jnp.tanh(
        jnp.dot(h_ref[...], p, preferred_element_type=_f32) / nrm)


def _score(h, pcol):
    return pl.pallas_call(
        _score_body,
        grid=(GRID,),
        in_specs=[
            pl.BlockSpec((RB, F), lambda i: (i, 0)),
            pl.BlockSpec((F, 1), lambda i: (0, 0)),
        ],
        out_specs=pl.BlockSpec((RB, 1), lambda i: (i, 0)),
        out_shape=jax.ShapeDtypeStruct((N, 1), _f32),
    )(h, pcol)


SROWS = 80
SCOLS = 125


def _topk_body(k, s_ref, selp_ref, sel_ref):
    bits = lax.bitcast_convert_type(s_ref[...], _i32)
    key = bits ^ ((bits >> 31) & jnp.int32(0x7FFFFFFF))
    imin = jnp.int32(-2147483648)
    key = jnp.where(selp_ref[...] > 0, key, imin)
    kf = _f32(k)

    t = jnp.int32(0)
    for i in range(31, -1, -1):
        bit = imin if i == 31 else jnp.int32(1 << i)
        cand_u = t | bit
        cand_s = cand_u ^ imin
        cnt = jnp.sum((key >= cand_s).astype(_f32))
        t = jnp.where(cnt >= kf, cand_u, t)
    thr = t ^ imin

    gt = key > thr
    eq = key == thr
    m = kf - jnp.sum(gt.astype(_f32))
    eqf = eq.astype(_f32)
    c128 = lax.broadcasted_iota(_i32, (128, 128), 0)
    r128 = lax.broadcasted_iota(_i32, (128, 128), 1)
    mtri = (c128 < r128).astype(_f32)
    inrow = jnp.dot(eqf, mtri, preferred_element_type=_f32)
    rowtot = jnp.sum(eqf, axis=1, keepdims=True)
    i80 = lax.broadcasted_iota(_i32, (SROWS, SROWS), 0)
    j80 = lax.broadcasted_iota(_i32, (SROWS, SROWS), 1)
    ltri = (j80 < i80).astype(_f32)
    rowpre = jnp.dot(ltri, rowtot, preferred_element_type=_f32)
    prefix = inrow + rowpre
    sel = gt | (eq & (prefix < m))
    sel_ref[...] = sel.astype(_i32)


def _topk(s2d, selp2d, k):
    return pl.pallas_call(
        functools.partial(_topk_body, k),
        out_shape=jax.ShapeDtypeStruct((SROWS, 128), _i32),
    )(s2d, selp2d)


def _final_body(h_ref, b_ref, sl_ref, w1_ref, b1_ref, w2_ref, b2_ref,
                o_ref, acc, cacc):
    i = pl.program_id(0)

    @pl.when(i == 0)
    def _():
        acc[...] = jnp.zeros_like(acc)
        cacc[...] = jnp.zeros_like(cacc)

    gi = lax.broadcasted_iota(_i32, (G, RB), 0)
    oh = (gi == b_ref[0]).astype(_f32) * sl_ref[0]
    acc[...] += jnp.dot(oh, h_ref[...], preferred_element_type=_f32)
    cacc[...] += jnp.sum(oh, axis=1, keepdims=True)

    @pl.when(i == GRID - 1)
    def _():
        g = acc[...] / jnp.maximum(cacc[...], 1.0)
        z = jnp.maximum(
            jnp.dot(g, w1_ref[...], preferred_element_type=_f32)
            + b1_ref[...], 0.0)
        o = jnp.dot(z, w2_ref[...], preferred_element_type=_f32) + b2_ref[...]
        o_ref[...] = 1.0 / (1.0 + jnp.exp(-o))


def _final(h5, batchrow, selrow, fcw1, fcb1, fcw2, fcb2):
    return pl.pallas_call(
        _final_body,
        grid=(GRID,),
        in_specs=[
            pl.BlockSpec((RB, F), lambda i: (i, 0)),
            pl.BlockSpec((1, 1, RB), lambda i: (i, 0, 0)),
            pl.BlockSpec((1, 1, RB), lambda i: (i, 0, 0)),
            pl.BlockSpec((F, F), lambda i: (0, 0)),
            pl.BlockSpec((1, F), lambda i: (0, 0)),
            pl.BlockSpec((F, 1), lambda i: (0, 0)),
            pl.BlockSpec((1, 1), lambda i: (0, 0)),
        ],
        out_specs=pl.BlockSpec((G, 1), lambda i: (0, 0)),
        out_shape=jax.ShapeDtypeStruct((G, 1), _f32),
        scratch_shapes=[
            pltpu.VMEM((G, F), _f32),
            pltpu.VMEM((G, 1), _f32),
        ],
    )(h5, batchrow, selrow, fcw1, fcb1, fcw2, fcb2)


def _sel_pieces(sel2d):
    sel_flat = sel2d[:, :SCOLS].reshape(N)
    sel_tab = jnp.pad(sel_flat, (0, NP - N))
    sel_col = sel_flat.astype(_f32).reshape(N, 1)
    return sel_tab, sel_col


def _score2d(scol):
    return jnp.pad(scol.reshape(SROWS, SCOLS), ((0, 0), (0, 128 - SCOLS)))


def kernel(x, edge_index, y, batch, W1, b1, W2, b2, W3, b3, W4, b4, W5, b5,
           p1, p2, fcW1, fcb1, fcW2, fcb2):
    del y
    src2d = jnp.pad(edge_index[:, 0].astype(_i32).reshape(ER, RW),
                    ((0, ERP - ER), (0, 0)))
    dst2d = jnp.pad(edge_index[:, 1].astype(_i32).reshape(ER, RW),
                    ((0, ERP - ER), (0, 0)), constant_values=TRASH)

    onesN = jnp.ones((N, 1), _f32)
    k1 = (N + 1) // 2
    k2 = (k1 + 1) // 2

    degp0 = _sc_deg0(dst2d)
    d00 = degp0[0, :N, 0].reshape(N, 1)
    d01 = degp0[1, :N, 0].reshape(N, 1)
    th1, dinv0 = _mm_scale(x, W1, d00, d01, onesN, onesN)
    P = _sc_agg(th1, src2d, dst2d)
    h1, th2 = _combine_mm(P[0, :N], P[1, :N], th1, dinv0,
                          b1.reshape(1, F), W2)
    P = _sc_agg(th2, src2d, dst2d)
    h2 = _combine(P[0, :N], P[1, :N], th2, dinv0, b2.reshape(1, F))

    s1 = _score(h2, p1.reshape(F, 1))
    sel1_2d = _topk(_score2d(s1), jnp.ones((SROWS, 128), _i32)
                    .at[:, SCOLS:].set(0), k1)
    sel1_tab, sel1_col = _sel_pieces(sel1_2d)

    dste1, degp1 = _sc_prep(src2d, dst2d, sel1_tab)
    d10 = degp1[0, :N, 0].reshape(N, 1)
    d11 = degp1[1, :N, 0].reshape(N, 1)
    th3, dinv1 = _mm_scale(h2, W3, d10, d11, s1, sel1_col)
    P = _sc_agg(th3, src2d, dste1)
    h3, th4 = _combine_mm(P[0, :N], P[1, :N], th3, dinv1,
                          b3.reshape(1, F), W4)
    P = _sc_agg(th4, src2d, dste1)
    h4 = _combine(P[0, :N], P[1, :N], th4, dinv1, b4.reshape(1, F))

    s2 = _score(h4, p2.reshape(F, 1))
    sel2_2d = _topk(_score2d(s2), sel1_2d, k2)
    sel2_tab, sel2_col = _sel_pieces(sel2_2d)

    dste2, degp2 = _sc_prep(src2d, dste1, sel2_tab)
    d20 = degp2[0, :N, 0].reshape(N, 1)
    d21 = degp2[1, :N, 0].reshape(N, 1)
    th5, dinv2 = _mm_scale(h4, W5, d20, d21, s2, sel2_col)
    P = _sc_agg(th5, src2d, dste2)
    h5 = _combine(P[0, :N], P[1, :N], th5, dinv2, b5.reshape(1, F))

    out = _final(h5, batch.astype(_i32).reshape(GRID, 1, RB),
                 sel2_col.reshape(GRID, 1, RB), fcW1, fcb1.reshape(1, F),
                 fcW2, fcb2.reshape(1, 1))
    return out.reshape(-1)


# speedup vs baseline: 14.7057x; 1.1965x over previous
import functools

import jax
import jax.numpy as jnp
from jax import lax
from jax.experimental import pallas as pl
from jax.experimental.pallas import tpu as pltpu
from jax.experimental.pallas import tpu_sc as plsc

N = 10000
E = 320000
G = 64
F = 64
NP = 10240
TRASH = N

RW = 128
ER = E // RW
ERP = 2560
NW = 32
RPW = ERP // NW
STRIPE = NP // 16

RB = 1000
GRID = N // RB

_f32 = jnp.float32
_i32 = jnp.int32

_mesh = plsc.VectorSubcoreMesh(core_axis_name="c", subcore_axis_name="s")
_sc_params = pltpu.CompilerParams(use_tc_tiling_on_sc=False,
                                  needs_layout_passes=False)


def _wid(cid, sid):
    return cid * 16 + sid


VPR = RW // 16


@functools.partial(
    pl.kernel,
    out_type=[
        jax.ShapeDtypeStruct((ERP, RW), _i32),
        jax.ShapeDtypeStruct((2, NP, 16), _f32),
    ],
    mesh=_mesh,
    compiler_params=_sc_params,
    scratch_types=[
        pltpu.VMEM((NP,), _i32),
        pltpu.VMEM((RPW, RW), _i32),
        pltpu.VMEM((RPW, RW), _i32),
        pltpu.VMEM((RPW, RW), _i32),
        pltpu.VMEM((RW, 16), _f32),
        pltpu.VMEM((STRIPE, 16), _f32),
        pltpu.VMEM_SHARED((NP, 16), _f32),
    ],
)
def _sc_prep(src_h, dstin_h, sel_h, dstm_h, degp_h,
             selv, srcv, dstv, dstmv, onesv, tmpv, acc):
    cid = lax.axis_index("c")
    sid = lax.axis_index("s")
    wid = _wid(cid, sid)

    pltpu.sync_copy(sel_h, selv)
    pltpu.sync_copy(src_h.at[pl.ds(wid * RPW, RPW)], srcv)
    pltpu.sync_copy(dstin_h.at[pl.ds(wid * RPW, RPW)], dstv)

    z16 = jnp.zeros((16,), _f32)
    trash16 = jnp.full((16,), TRASH, _i32)
    one0 = jnp.where(lax.iota(_i32, 16) == 0, 1.0, 0.0).astype(_f32)

    def zstripe(r, _):
        tmpv[r, pl.ds(0, 16)] = z16
        return 0

    lax.fori_loop(0, STRIPE, zstripe, 0)
    pltpu.sync_copy(tmpv, acc.at[pl.ds(sid * STRIPE, STRIPE)])

    def onesinit(r, _):
        onesv[r, pl.ds(0, 16)] = one0
        return 0

    lax.fori_loop(0, RW, onesinit, 0)
    plsc.subcore_barrier()

    def row(r, _):
        def vec(i, _):
            s16 = srcv[r, pl.ds(i * 16, 16)]
            d16 = dstv[r, pl.ds(i * 16, 16)]
            ssel = plsc.load_gather(selv, [s16])
            dsel = plsc.load_gather(selv, [d16])
            valid = (ssel > 0) & (dsel > 0)
            dstmv[r, pl.ds(i * 16, 16)] = jnp.where(valid, d16, trash16)
            return 0

        lax.fori_loop(0, VPR, vec, 0)
        pltpu.sync_copy(onesv, acc.at[dstmv.at[r]], add=True)
        return 0

    lax.fori_loop(0, RPW, row, 0)
    pltpu.sync_copy(dstmv, dstm_h.at[pl.ds(wid * RPW, RPW)])

    plsc.subcore_barrier()
    off = sid * STRIPE
    pltpu.sync_copy(acc.at[pl.ds(off, STRIPE)], tmpv)
    pltpu.sync_copy(tmpv, degp_h.at[cid, pl.ds(off, STRIPE)])


MBUF = 8
NGRP = RPW // MBUF


@functools.partial(
    pl.kernel,
    out_type=jax.ShapeDtypeStruct((2, NP, F), _f32),
    mesh=_mesh,
    compiler_params=_sc_params,
    scratch_types=(
        [
            pltpu.VMEM((RPW, RW), _i32),
            pltpu.VMEM((RPW, RW), _i32),
        ]
        + [pltpu.VMEM((RW, F), _f32) for _ in range(MBUF)]
        + [pltpu.SemaphoreType.DMA for _ in range(MBUF)]
        + [pltpu.VMEM_SHARED((NP, F), _f32)]
    ),
)
def _sc_agg(tab_h, src_h, dste_h, p_h, srcv, dstev, *rest):
    bufs = rest[:MBUF]
    gsem = rest[MBUF:2 * MBUF]
    acc = rest[2 * MBUF]
    cid = lax.axis_index("c")
    sid = lax.axis_index("s")
    wid = _wid(cid, sid)

    def zfill(i, _):
        r = i // (F // 16)
        cidx = (i % (F // 16)) * 16
        bufs[0][r, pl.ds(cidx, 16)] = jnp.zeros((16,), _f32)
        return 0

    lax.fori_loop(0, RW * (F // 16), zfill, 0)

    def zstripe(t, _):
        pltpu.sync_copy(bufs[0], acc.at[pl.ds(sid * STRIPE + t * RW, RW)])
        return 0

    lax.fori_loop(0, STRIPE // RW, zstripe, 0)

    pltpu.sync_copy(src_h.at[pl.ds(wid * RPW, RPW)], srcv)
    pltpu.sync_copy(dste_h.at[pl.ds(wid * RPW, RPW)], dstev)

    for b in range(MBUF):
        pltpu.async_copy(tab_h.at[srcv.at[b]], bufs[b], gsem[b])
    plsc.subcore_barrier()

    def group(g, _):
        for b in range(MBUF):
            r = g * MBUF + b
            pltpu.make_async_copy(tab_h.at[srcv.at[r]], bufs[b],
                                  gsem[b]).wait()
            pltpu.sync_copy(bufs[b], acc.at[dstev.at[r]], add=True)

            @pl.when(r + MBUF < RPW)
            def _():
                pltpu.async_copy(tab_h.at[srcv.at[r + MBUF]], bufs[b],
                                 gsem[b])

        return 0

    lax.fori_loop(0, NGRP, group, 0)
    plsc.subcore_barrier()

    def out(t, _):
        off = sid * STRIPE + t * RW
        pltpu.sync_copy(acc.at[pl.ds(off, RW)], bufs[0])
        pltpu.sync_copy(bufs[0], p_h.at[cid, pl.ds(off, RW)])
        return 0

    lax.fori_loop(0, STRIPE // RW, out, 0)


def _mm_scale_body(x_ref, w_ref, d0_ref, d1_ref, sc_ref, sl_ref, th_ref,
                   dinv_ref):
    deg = 1.0 + d0_ref[...] + d1_ref[...]
    dinv = lax.rsqrt(deg)
    xs = x_ref[...] * (sc_ref[...] * sl_ref[...])
    t = jnp.dot(xs, w_ref[...], preferred_element_type=_f32)
    th_ref[...] = t * dinv
    dinv_ref[...] = dinv


def _mm_scale(x, w, d0, d1, scol, slcol):
    k = x.shape[1]
    return pl.pallas_call(
        _mm_scale_body,
        grid=(GRID,),
        in_specs=[
            pl.BlockSpec((RB, k), lambda i: (i, 0)),
            pl.BlockSpec((k, F), lambda i: (0, 0)),
            pl.BlockSpec((RB, 1), lambda i: (i, 0)),
            pl.BlockSpec((RB, 1), lambda i: (i, 0)),
            pl.BlockSpec((RB, 1), lambda i: (i, 0)),
            pl.BlockSpec((RB, 1), lambda i: (i, 0)),
        ],
        out_specs=[
            pl.BlockSpec((RB, F), lambda i: (i, 0)),
            pl.BlockSpec((RB, 1), lambda i: (i, 0)),
        ],
        out_shape=[
            jax.ShapeDtypeStruct((N, F), _f32),
            jax.ShapeDtypeStruct((N, 1), _f32),
        ],
    )(x, w, d0, d1, scol, slcol)


def _combine_mm_body(p0_ref, p1_ref, th_ref, dinv_ref, b_ref, w_ref, h_ref,
                     th2_ref):
    dinv = dinv_ref[...]
    h = jnp.maximum(
        dinv * (p0_ref[...] + p1_ref[...] + th_ref[...]) + b_ref[...], 0.0)
    h_ref[...] = h
    th2_ref[...] = jnp.dot(h, w_ref[...], preferred_element_type=_f32) * dinv


def _combine_mm(p0, p1, th, dinv, b, w):
    return pl.pallas_call(
        _combine_mm_body,
        grid=(GRID,),
        in_specs=[
            pl.BlockSpec((RB, F), lambda i: (i, 0)),
            pl.BlockSpec((RB, F), lambda i: (i, 0)),
            pl.BlockSpec((RB, F), lambda i: (i, 0)),
            pl.BlockSpec((RB, 1), lambda i: (i, 0)),
            pl.BlockSpec((1, F), lambda i: (0, 0)),
            pl.BlockSpec((F, F), lambda i: (0, 0)),
        ],
        out_specs=[
            pl.BlockSpec((RB, F), lambda i: (i, 0)),
            pl.BlockSpec((RB, F), lambda i: (i, 0)),
        ],
        out_shape=[
            jax.ShapeDtypeStruct((N, F), _f32),
            jax.ShapeDtypeStruct((N, F), _f32),
        ],
    )(p0, p1, th, dinv, b, w)


def _combine_body(p0_ref, p1_ref, th_ref, dinv_ref, b_ref, h_ref):
    h_ref[...] = jnp.maximum(
        dinv_ref[...] * (p0_ref[...] + p1_ref[...] + th_ref[...])
        + b_ref[...], 0.0)


def _combine(p0, p1, th, dinv, b):
    return pl.pallas_call(
        _combine_body,
        grid=(GRID,),
        in_specs=[
            pl.BlockSpec((RB, F), lambda i: (i, 0)),
            pl.BlockSpec((RB, F), lambda i: (i, 0)),
            pl.BlockSpec((RB, F), lambda i: (i, 0)),
            pl.BlockSpec((RB, 1), lambda i: (i, 0)),
            pl.BlockSpec((1, F), lambda i: (0, 0)),
        ],
        out_specs=pl.BlockSpec((RB, F), lambda i: (i, 0)),
        out_shape=jax.ShapeDtypeStruct((N, F), _f32),
    )(p0, p1, th, dinv, b)


def _score_body(h_ref, p_ref, s_ref):
    p = p_ref[...]
    nrm = jnp.sqrt(jnp.sum(p * p)) + 1e-16
    s_ref[...] = jnp.tanh(
        jnp.dot(h_ref[...], p, preferred_element_type=_f32) / nrm)


def _score(h, pcol):
    return pl.pallas_call(
        _score_body,
        grid=(GRID,),
        in_specs=[
            pl.BlockSpec((RB, F), lambda i: (i, 0)),
            pl.BlockSpec((F, 1), lambda i: (0, 0)),
        ],
        out_specs=pl.BlockSpec((RB, 1), lambda i: (i, 0)),
        out_shape=jax.ShapeDtypeStruct((N, 1), _f32),
    )(h, pcol)


SROWS = 80
SCOLS = 125


def _topk_body(k, s_ref, selp_ref, sel_ref):
    bits = lax.bitcast_convert_type(s_ref[...], _i32)
    key = bits ^ ((bits >> 31) & jnp.int32(0x7FFFFFFF))
    imin = jnp.int32(-2147483648)
    key = jnp.where(selp_ref[...] > 0, key, imin)
    kf = _f32(k)

    t = jnp.int32(0)
    for i in range(31, -1, -1):
        bit = imin if i == 31 else jnp.int32(1 << i)
        cand_u = t | bit
        cand_s = cand_u ^ imin
        cnt = jnp.sum((key >= cand_s).astype(_f32))
        t = jnp.where(cnt >= kf, cand_u, t)
    thr = t ^ imin

    gt = key > thr
    eq = key == thr
    m = kf - jnp.sum(gt.astype(_f32))
    eqf = eq.astype(_f32)
    c128 = lax.broadcasted_iota(_i32, (128, 128), 0)
    r128 = lax.broadcasted_iota(_i32, (128, 128), 1)
    mtri = (c128 < r128).astype(_f32)
    inrow = jnp.dot(eqf, mtri, preferred_element_type=_f32)
    rowtot = jnp.sum(eqf, axis=1, keepdims=True)
    i80 = lax.broadcasted_iota(_i32, (SROWS, SROWS), 0)
    j80 = lax.broadcasted_iota(_i32, (SROWS, SROWS), 1)
    ltri = (j80 < i80).astype(_f32)
    rowpre = jnp.dot(ltri, rowtot, preferred_element_type=_f32)
    prefix = inrow + rowpre
    sel = gt | (eq & (prefix < m))
    sel_ref[...] = sel.astype(_i32)


def _topk(s2d, selp2d, k):
    return pl.pallas_call(
        functools.partial(_topk_body, k),
        out_shape=jax.ShapeDtypeStruct((SROWS, 128), _i32),
    )(s2d, selp2d)


def _final_body(h_ref, b_ref, sl_ref, w1_ref, b1_ref, w2_ref, b2_ref,
                o_ref, acc, cacc):
    i = pl.program_id(0)

    @pl.when(i == 0)
    def _():
        acc[...] = jnp.zeros_like(acc)
        cacc[...] = jnp.zeros_like(cacc)

    gi = lax.broadcasted_iota(_i32, (G, RB), 0)
    oh = (gi == b_ref[0]).astype(_f32) * sl_ref[0]
    acc[...] += jnp.dot(oh, h_ref[...], preferred_element_type=_f32)
    cacc[...] += jnp.sum(oh, axis=1, keepdims=True)

    @pl.when(i == GRID - 1)
    def _():
        g = acc[...] / jnp.maximum(cacc[...], 1.0)
        z = jnp.maximum(
            jnp.dot(g, w1_ref[...], preferred_element_type=_f32)
            + b1_ref[...], 0.0)
        o = jnp.dot(z, w2_ref[...], preferred_element_type=_f32) + b2_ref[...]
        o_ref[...] = 1.0 / (1.0 + jnp.exp(-o))


def _final(h5, batchrow, selrow, fcw1, fcb1, fcw2, fcb2):
    return pl.pallas_call(
        _final_body,
        grid=(GRID,),
        in_specs=[
            pl.BlockSpec((RB, F), lambda i: (i, 0)),
            pl.BlockSpec((1, 1, RB), lambda i: (i, 0, 0)),
            pl.BlockSpec((1, 1, RB), lambda i: (i, 0, 0)),
            pl.BlockSpec((F, F), lambda i: (0, 0)),
            pl.BlockSpec((1, F), lambda i: (0, 0)),
            pl.BlockSpec((F, 1), lambda i: (0, 0)),
            pl.BlockSpec((1, 1), lambda i: (0, 0)),
        ],
        out_specs=pl.BlockSpec((G, 1), lambda i: (0, 0)),
        out_shape=jax.ShapeDtypeStruct((G, 1), _f32),
        scratch_shapes=[
            pltpu.VMEM((G, F), _f32),
            pltpu.VMEM((G, 1), _f32),
        ],
    )(h5, batchrow, selrow, fcw1, fcb1, fcw2, fcb2)


def _sel_pieces(sel2d):
    sel_flat = sel2d[:, :SCOLS].reshape(N)
    sel_tab = jnp.pad(sel_flat, (0, NP - N))
    sel_col = sel_flat.astype(_f32).reshape(N, 1)
    return sel_tab, sel_col


def _score2d(scol):
    return jnp.pad(scol.reshape(SROWS, SCOLS), ((0, 0), (0, 128 - SCOLS)))


def kernel(x, edge_index, y, batch, W1, b1, W2, b2, W3, b3, W4, b4, W5, b5,
           p1, p2, fcW1, fcb1, fcW2, fcb2):
    del y
    src2d = jnp.pad(edge_index[:, 0].astype(_i32).reshape(ER, RW),
                    ((0, ERP - ER), (0, 0)))
    dst2d = jnp.pad(edge_index[:, 1].astype(_i32).reshape(ER, RW),
                    ((0, ERP - ER), (0, 0)), constant_values=TRASH)

    onesN = jnp.ones((N, 1), _f32)
    k1 = (N + 1) // 2
    k2 = (k1 + 1) // 2

    sel0_tab = jnp.pad(jnp.ones((N,), _i32), (0, NP - N))
    dste0, degp0 = _sc_prep(src2d, dst2d, sel0_tab)
    d00 = degp0[0, :N, 0].reshape(N, 1)
    d01 = degp0[1, :N, 0].reshape(N, 1)
    th1, dinv0 = _mm_scale(x, W1, d00, d01, onesN, onesN)
    P = _sc_agg(th1, src2d, dste0)
    h1, th2 = _combine_mm(P[0, :N], P[1, :N], th1, dinv0,
                          b1.reshape(1, F), W2)
    P = _sc_agg(th2, src2d, dste0)
    h2 = _combine(P[0, :N], P[1, :N], th2, dinv0, b2.reshape(1, F))

    s1 = _score(h2, p1.reshape(F, 1))
    sel1_2d = _topk(_score2d(s1), jnp.ones((SROWS, 128), _i32)
                    .at[:, SCOLS:].set(0), k1)
    sel1_tab, sel1_col = _sel_pieces(sel1_2d)

    dste1, degp1 = _sc_prep(src2d, dst2d, sel1_tab)
    d10 = degp1[0, :N, 0].reshape(N, 1)
    d11 = degp1[1, :N, 0].reshape(N, 1)
    th3, dinv1 = _mm_scale(h2, W3, d10, d11, s1, sel1_col)
    P = _sc_agg(th3, src2d, dste1)
    h3, th4 = _combine_mm(P[0, :N], P[1, :N], th3, dinv1,
                          b3.reshape(1, F), W4)
    P = _sc_agg(th4, src2d, dste1)
    h4 = _combine(P[0, :N], P[1, :N], th4, dinv1, b4.reshape(1, F))

    s2 = _score(h4, p2.reshape(F, 1))
    sel2_2d = _topk(_score2d(s2), sel1_2d, k2)
    sel2_tab, sel2_col = _sel_pieces(sel2_2d)

    dste2, degp2 = _sc_prep(src2d, dst2d, sel2_tab)
    d20 = degp2[0, :N, 0].reshape(N, 1)
    d21 = degp2[1, :N, 0].reshape(N, 1)
    th5, dinv2 = _mm_scale(h4, W5, d20, d21, s2, sel2_col)
    P = _sc_agg(th5, src2d, dste2)
    h5 = _combine(P[0, :N], P[1, :N], th5, dinv2, b5.reshape(1, F))

    out = _final(h5, batch.astype(_i32).reshape(GRID, 1, RB),
                 sel2_col.reshape(GRID, 1, RB), fcW1, fcb1.reshape(1, F),
                 fcW2, fcb2.reshape(1, 1))
    return out.reshape(-1)
